# baseline (speedup 1.0000x reference)
import jax
import jax.numpy as jnp
from jax import lax
from jax.experimental import pallas as pl
from jax.experimental.pallas import tpu as pltpu
from jax.experimental.pallas import tpu_sc as plsc

N = 10000
D = 128
E = 320000
NPAD = 10240
C = 128
NCHUNK = E // C
G = 8
NGROUP = NCHUNK // G
NTAIL = NCHUNK - NGROUP * G
NW = 32
MAXG_W = -(-NGROUP // NW)
EXTRA = NGROUP - (NGROUP // NW) * NW
SLICE = NPAD // 16


def _whT_body(w0, w1, w2, b0, b1, b2, feat_ref, o0, o1, o2):
    w3 = jnp.concatenate([w0[...], w1[...], w2[...]], axis=0)
    res = lax.dot_general(w3, feat_ref[...], (((1,), (1,)), ((), ())),
                          preferred_element_type=jnp.float32)
    o0[...] = res[0:1] + b0[...]
    o1[...] = res[1:2] + b1[...]
    o2[...] = res[2:3] + b2[...]


def _whT(W_f, b_f, W_c, b_c, W_l, b_l, feat):
    out = jax.ShapeDtypeStruct((1, N), jnp.float32)
    return pl.pallas_call(
        _whT_body,
        out_shape=(out, out, out),
    )(W_f, W_c, W_l, b_f.reshape(1, 1), b_c.reshape(1, 1), b_l.reshape(1, 1),
      feat)


def _sc_body(wh0, wh1, wh2_, e0, e1, e2,
             out_hbm,
             srcA, dstA, valA, srcB, dstB, valB,
             pair_loc, sums_loc, cnt_loc, idxz, zrows,
             whv0, whv1, whv2,
             a0, a1, a2, sem_ld, sem_st, sem_s):
    cid = lax.axis_index("c")
    sid = lax.axis_index("s")
    wid = cid * 16 + sid

    for wh_h, wh_v in ((wh0, whv0), (wh1, whv1), (wh2_, whv2)):
        pltpu.async_copy(wh_h, wh_v, sem_st)

    iota16 = lax.iota(jnp.int32, 16)
    zeros16i = jnp.zeros((16,), jnp.int32)
    ones16i = jnp.ones((16,), jnp.int32)
    ones16f = jnp.ones((16,), jnp.float32)

    for val_b in (valA, valB):
        for m in range(G * C // 16):
            plsc.store_scatter(val_b, [m * 16 + iota16, ones16i], ones16f)

    zeros16f = jnp.zeros((16,), jnp.float32)
    for i in range(C // 16):
        plsc.store_scatter(zrows, [i * 16 + iota16, zeros16i], zeros16f)
        plsc.store_scatter(zrows, [i * 16 + iota16, ones16i], zeros16f)
    for r in range(SLICE // C):
        for i in range(C // 16):
            idxz[r, pl.ds(i * 16, 16)] = sid * SLICE + r * C + i * 16 + iota16

    accs = (a0, a1, a2)
    for a in accs:
        for r in range(SLICE // C):
            pltpu.async_copy(zrows, a.at[idxz.at[r]], sem_s)
    for a in accs:
        for r in range(SLICE // C):
            pltpu.make_async_copy(zrows, a.at[idxz.at[r]], sem_s).wait()
    for wh_h, wh_v in ((wh0, whv0), (wh1, whv1), (wh2_, whv2)):
        pltpu.make_async_copy(wh_h, wh_v, sem_st).wait()
    plsc.subcore_barrier()

    ng = jnp.where(wid < EXTRA, MAXG_W, MAXG_W - 1)

    bufs = ((srcA, dstA, valA), (srcB, dstB, valB))

    for et, (e3d, wh_v, acc) in enumerate(
            ((e0, whv0, a0),
             (e1, whv1, a1),
             (e2, whv2, a2))):

        def fire_loads(k, sl):
            src_b, dst_b, _ = bufs[sl]
            g = wid + NW * k
            pltpu.async_copy(e3d.at[0, pl.ds(g * G, G)], src_b, sem_ld)
            pltpu.async_copy(e3d.at[1, pl.ds(g * G, G)], dst_b, sem_ld)

        def drain_loads(k, sl):
            src_b, dst_b, _ = bufs[sl]
            g = wid + NW * k
            pltpu.make_async_copy(e3d.at[0, pl.ds(g * G, G)], src_b,
                                  sem_ld).wait()
            pltpu.make_async_copy(e3d.at[1, pl.ds(g * G, G)], dst_b,
                                  sem_ld).wait()

        def compute_vals(sl, nchunks=G):
            src_b, _, val_b = bufs[sl]
            for j in range(nchunks):
                for i in range(C // 16):
                    idx16 = src_b[j, pl.ds(i * 16, 16)]
                    vals = plsc.load_gather(wh_v, [zeros16i, idx16])
                    plsc.store_scatter(
                        val_b, [j * C + i * 16 + iota16, zeros16i], vals)

        def fire_scatters(sl, nchunks=G):
            _, dst_b, val_b = bufs[sl]
            for j in range(nchunks):
                pltpu.async_copy(val_b.at[pl.ds(j * C, C), :],
                                 acc.at[dst_b.at[j]], sem_s, add=True)

        def drain_scatters(sl, nchunks=G):
            _, dst_b, val_b = bufs[sl]
            for j in range(nchunks):
                pltpu.make_async_copy(val_b.at[pl.ds(j * C, C), :],
                                      acc.at[dst_b.at[j]], sem_s).wait()

        fire_loads(0, 0)

        def pbody(p, carry):
            @pl.when(2 * p < ng)
            def _():
                drain_loads(2 * p, 0)
                compute_vals(0)
                @pl.when(2 * p - 1 >= 0)
                def _():
                    drain_scatters(1)
                @pl.when(2 * p + 1 < ng)
                def _():
                    fire_loads(2 * p + 1, 1)
                fire_scatters(0)
            @pl.when(2 * p + 1 < ng)
            def _():
                drain_loads(2 * p + 1, 1)
                compute_vals(1)
                drain_scatters(0)
                @pl.when(2 * p + 2 < ng)
                def _():
                    fire_loads(2 * p + 2, 0)
                fire_scatters(1)
            return carry

        lax.fori_loop(0, MAXG_W // 2, pbody, 0)
        @pl.when(ng == MAXG_W)
        def _():
            drain_scatters(1)
        @pl.when(ng == MAXG_W - 1)
        def _():
            drain_scatters(0)

        @pl.when(wid == EXTRA + et)
        def _():
            pltpu.sync_copy(e3d.at[0, pl.ds(NGROUP * G, NTAIL)],
                            srcA.at[pl.ds(0, NTAIL)])
            pltpu.sync_copy(e3d.at[1, pl.ds(NGROUP * G, NTAIL)],
                            dstA.at[pl.ds(0, NTAIL)])
            compute_vals(0, NTAIL)
            fire_scatters(0, NTAIL)
            drain_scatters(0, NTAIL)

    plsc.subcore_barrier()
    for et, a in enumerate(accs):
        pltpu.sync_copy(a.at[pl.ds(sid * SLICE, SLICE), :], pair_loc)
        for i in range(SLICE // 16):
            rows = i * 16 + iota16
            sums_loc[pl.ds(i * 16, 16)] = plsc.load_gather(
                pair_loc, [rows, zeros16i])
            cnt_loc[pl.ds(i * 16, 16)] = plsc.load_gather(
                pair_loc, [rows, ones16i])
        off_s = (cid * 6 + et) * NPAD + sid * SLICE
        off_c = (cid * 6 + 3 + et) * NPAD + sid * SLICE
        pltpu.sync_copy(sums_loc, out_hbm.at[pl.ds(off_s, SLICE)])
        pltpu.sync_copy(cnt_loc, out_hbm.at[pl.ds(off_c, SLICE)])


def _sc_scatter(wh_list, edge_list):
    mesh = plsc.VectorSubcoreMesh(core_axis_name="c", subcore_axis_name="s")
    kfn = pl.kernel(
        _sc_body,
        out_type=jax.ShapeDtypeStruct((12 * NPAD,), jnp.float32),
        mesh=mesh,
        compiler_params=pltpu.CompilerParams(needs_layout_passes=False,
                                             use_tc_tiling_on_sc=False),
        scratch_types=[
            pltpu.VMEM((G, C), jnp.int32),
            pltpu.VMEM((G, C), jnp.int32),
            pltpu.VMEM((G * C, 2), jnp.float32),
            pltpu.VMEM((G, C), jnp.int32),
            pltpu.VMEM((G, C), jnp.int32),
            pltpu.VMEM((G * C, 2), jnp.float32),
            pltpu.VMEM((SLICE, 2), jnp.float32),
            pltpu.VMEM((SLICE,), jnp.float32),
            pltpu.VMEM((SLICE,), jnp.float32),
            pltpu.VMEM((SLICE // C, C), jnp.int32),
            pltpu.VMEM((C, 2), jnp.float32),
            pltpu.VMEM((1, N), jnp.float32),
            pltpu.VMEM((1, N), jnp.float32),
            pltpu.VMEM((1, N), jnp.float32),
            pltpu.VMEM_SHARED((NPAD, 2), jnp.float32),
            pltpu.VMEM_SHARED((NPAD, 2), jnp.float32),
            pltpu.VMEM_SHARED((NPAD, 2), jnp.float32),
            pltpu.SemaphoreType.DMA,
            pltpu.SemaphoreType.DMA,
            pltpu.SemaphoreType.DMA,
        ],
    )
    return kfn(wh_list[0], wh_list[1], wh_list[2],
               edge_list[0], edge_list[1], edge_list[2])


def _combine_body(p_ref, o_ref):
    p = p_ref[...].reshape(12, NPAD)
    sums = p[0:3] + p[6:9]
    cnt = p[3:6] + p[9:12]
    h = jnp.sum(jnp.where(cnt > 0, sums / jnp.maximum(cnt, 1.0), 0.0), axis=0)
    o_ref[...] = h[None, :]


def _combine(p):
    return pl.pallas_call(
        _combine_body,
        out_shape=jax.ShapeDtypeStruct((1, NPAD), jnp.float32),
    )(p)


@jax.jit
def kernel(feat, edge_index_follows, edge_index_connects, edge_index_links,
           W_follows, b_follows, W_connects, b_connects, W_links, b_links):
    wh_list = _whT(W_follows, b_follows, W_connects, b_connects,
                   W_links, b_links, feat)

    edge_list = [e.reshape(2, NCHUNK, C) for e in
                 (edge_index_follows, edge_index_connects, edge_index_links)]

    partials = _sc_scatter(wh_list, edge_list)

    out1 = _combine(partials)
    return out1[0, :N].reshape(N, 1)


# speedup vs baseline: 1.1136x; 1.1136x over previous
import jax
import jax.numpy as jnp
from jax import lax
from jax.experimental import pallas as pl
from jax.experimental.pallas import tpu as pltpu
from jax.experimental.pallas import tpu_sc as plsc

N = 10000
D = 128
E = 320000
NPAD = 10240
C = 128
NCHUNK = E // C
G = 8
NGROUP = NCHUNK // G
NTAIL = NCHUNK - NGROUP * G
NW = 32
MAXG_W = -(-NGROUP // NW)
EXTRA = NGROUP - (NGROUP // NW) * NW
SLICE = NPAD // 16


def _whT_body(w0, w1, w2, b0, b1, b2, feat_ref, o0, o1, o2):
    w3 = jnp.concatenate([w0[...], w1[...], w2[...]], axis=0)
    res = lax.dot_general(w3, feat_ref[...], (((1,), (1,)), ((), ())),
                          preferred_element_type=jnp.float32)
    o0[...] = res[0:1] + b0[...]
    o1[...] = res[1:2] + b1[...]
    o2[...] = res[2:3] + b2[...]


def _whT(W_f, b_f, W_c, b_c, W_l, b_l, feat):
    out = jax.ShapeDtypeStruct((1, N), jnp.float32)
    return pl.pallas_call(
        _whT_body,
        out_shape=(out, out, out),
    )(W_f, W_c, W_l, b_f.reshape(1, 1), b_c.reshape(1, 1), b_l.reshape(1, 1),
      feat)


def _sc_body(wh0, wh1, wh2_, e0, e1, e2,
             out_hbm,
             srcA, dstA, valA, srcB, dstB, valB, ones_buf, zero_buf,
             whv0, whv1, whv2,
             s0, s1, s2, c0, c1, c2, sem_ld, sem_st, sem_s):
    cid = lax.axis_index("c")
    sid = lax.axis_index("s")
    wid = cid * 16 + sid

    for wh_h, wh_v in ((wh0, whv0), (wh1, whv1), (wh2_, whv2)):
        pltpu.async_copy(wh_h, wh_v, sem_st)

    ones16 = jnp.ones((16,), jnp.float32)
    zeros16 = jnp.zeros((16,), jnp.float32)
    for i in range(C // 16):
        ones_buf[pl.ds(i * 16, 16)] = ones16
    for i in range(SLICE // 16):
        zero_buf[pl.ds(i * 16, 16)] = zeros16

    accs = (s0, s1, s2, c0, c1, c2)
    for a in accs:
        pltpu.sync_copy(zero_buf, a.at[pl.ds(sid * SLICE, SLICE)])
    for wh_h, wh_v in ((wh0, whv0), (wh1, whv1), (wh2_, whv2)):
        pltpu.make_async_copy(wh_h, wh_v, sem_st).wait()
    plsc.subcore_barrier()

    ng = jnp.where(wid < EXTRA, MAXG_W, MAXG_W - 1)

    bufs = ((srcA, dstA, valA), (srcB, dstB, valB))
    zeros16i = jnp.zeros((16,), jnp.int32)

    for et, (e3d, wh_v, acc_s, acc_c) in enumerate(
            ((e0, whv0, s0, c0),
             (e1, whv1, s1, c1),
             (e2, whv2, s2, c2))):

        def fire_loads(k, sl):
            src_b, dst_b, _ = bufs[sl]
            g = wid + NW * k
            pltpu.async_copy(e3d.at[0, pl.ds(g * G, G)], src_b, sem_ld)
            pltpu.async_copy(e3d.at[1, pl.ds(g * G, G)], dst_b, sem_ld)

        def drain_loads(k, sl):
            src_b, dst_b, _ = bufs[sl]
            g = wid + NW * k
            pltpu.make_async_copy(e3d.at[0, pl.ds(g * G, G)], src_b,
                                  sem_ld).wait()
            pltpu.make_async_copy(e3d.at[1, pl.ds(g * G, G)], dst_b,
                                  sem_ld).wait()

        def compute_vals(sl, nchunks=G):
            src_b, _, val_b = bufs[sl]
            for j in range(nchunks):
                for i in range(C // 16):
                    idx16 = src_b[j, pl.ds(i * 16, 16)]
                    val_b[j, pl.ds(i * 16, 16)] = plsc.load_gather(
                        wh_v, [zeros16i, idx16])

        def fire_scatters(sl, nchunks=G):
            _, dst_b, val_b = bufs[sl]
            for j in range(nchunks):
                pltpu.async_copy(val_b.at[j], acc_s.at[dst_b.at[j]],
                                 sem_s, add=True)
                pltpu.async_copy(ones_buf, acc_c.at[dst_b.at[j]],
                                 sem_s, add=True)

        def drain_scatters(sl, nchunks=G):
            _, dst_b, val_b = bufs[sl]
            for j in range(nchunks):
                pltpu.make_async_copy(val_b.at[j], acc_s.at[dst_b.at[j]],
                                      sem_s).wait()
                pltpu.make_async_copy(ones_buf, acc_c.at[dst_b.at[j]],
                                      sem_s).wait()

        fire_loads(0, 0)

        def pbody(p, carry):
            @pl.when(2 * p < ng)
            def _():
                drain_loads(2 * p, 0)
                compute_vals(0)
                @pl.when(2 * p - 1 >= 0)
                def _():
                    drain_scatters(1)
                @pl.when(2 * p + 1 < ng)
                def _():
                    fire_loads(2 * p + 1, 1)
                fire_scatters(0)
            @pl.when(2 * p + 1 < ng)
            def _():
                drain_loads(2 * p + 1, 1)
                compute_vals(1)
                drain_scatters(0)
                @pl.when(2 * p + 2 < ng)
                def _():
                    fire_loads(2 * p + 2, 0)
                fire_scatters(1)
            return carry

        lax.fori_loop(0, MAXG_W // 2, pbody, 0)
        @pl.when(ng == MAXG_W)
        def _():
            drain_scatters(1)
        @pl.when(ng == MAXG_W - 1)
        def _():
            drain_scatters(0)

        @pl.when(wid == EXTRA + et)
        def _():
            pltpu.sync_copy(e3d.at[0, pl.ds(NGROUP * G, NTAIL)],
                            srcA.at[pl.ds(0, NTAIL)])
            pltpu.sync_copy(e3d.at[1, pl.ds(NGROUP * G, NTAIL)],
                            dstA.at[pl.ds(0, NTAIL)])
            compute_vals(0, NTAIL)
            fire_scatters(0, NTAIL)
            drain_scatters(0, NTAIL)

    plsc.subcore_barrier()
    for k, a in enumerate(accs):
        off = (cid * 6 + k) * NPAD + sid * SLICE
        pltpu.sync_copy(a.at[pl.ds(sid * SLICE, SLICE)],
                        out_hbm.at[pl.ds(off, SLICE)])


def _sc_scatter(wh_list, edge_list):
    mesh = plsc.VectorSubcoreMesh(core_axis_name="c", subcore_axis_name="s")
    kfn = pl.kernel(
        _sc_body,
        out_type=jax.ShapeDtypeStruct((12 * NPAD,), jnp.float32),
        mesh=mesh,
        compiler_params=pltpu.CompilerParams(needs_layout_passes=False),
        scratch_types=[
            pltpu.VMEM((G, C), jnp.int32),
            pltpu.VMEM((G, C), jnp.int32),
            pltpu.VMEM((G, C), jnp.float32),
            pltpu.VMEM((G, C), jnp.int32),
            pltpu.VMEM((G, C), jnp.int32),
            pltpu.VMEM((G, C), jnp.float32),
            pltpu.VMEM((C,), jnp.float32),
            pltpu.VMEM((SLICE,), jnp.float32),
            pltpu.VMEM((1, N), jnp.float32),
            pltpu.VMEM((1, N), jnp.float32),
            pltpu.VMEM((1, N), jnp.float32),
            pltpu.VMEM_SHARED((NPAD,), jnp.float32),
            pltpu.VMEM_SHARED((NPAD,), jnp.float32),
            pltpu.VMEM_SHARED((NPAD,), jnp.float32),
            pltpu.VMEM_SHARED((NPAD,), jnp.float32),
            pltpu.VMEM_SHARED((NPAD,), jnp.float32),
            pltpu.VMEM_SHARED((NPAD,), jnp.float32),
            pltpu.SemaphoreType.DMA,
            pltpu.SemaphoreType.DMA,
            pltpu.SemaphoreType.DMA,
        ],
    )
    return kfn(wh_list[0], wh_list[1], wh_list[2],
               edge_list[0], edge_list[1], edge_list[2])


def _combine_body(p_ref, o_ref):
    p = p_ref[...].reshape(12, NPAD)
    sums = p[0:3] + p[6:9]
    cnt = p[3:6] + p[9:12]
    h = jnp.sum(jnp.where(cnt > 0, sums / jnp.maximum(cnt, 1.0), 0.0), axis=0)
    o_ref[...] = h[None, :]


def _combine(p):
    return pl.pallas_call(
        _combine_body,
        out_shape=jax.ShapeDtypeStruct((1, NPAD), jnp.float32),
    )(p)


@jax.jit
def kernel(feat, edge_index_follows, edge_index_connects, edge_index_links,
           W_follows, b_follows, W_connects, b_connects, W_links, b_links):
    wh_list = _whT(W_follows, b_follows, W_connects, b_connects,
                   W_links, b_links, feat)

    edge_list = [e.reshape(2, NCHUNK, C) for e in
                 (edge_index_follows, edge_index_connects, edge_index_links)]

    partials = _sc_scatter(wh_list, edge_list)

    out1 = _combine(partials)
    return out1[0, :N].reshape(N, 1)
